# baseline (speedup 1.0000x reference)
import jax
import jax.numpy as jnp
from jax.experimental import pallas as pl
from jax.experimental.pallas import tpu as pltpu

_K = 5
_PS = 3
_WIN = 12
_N1 = 30
_M1 = 62
_T = 31
_CE = 96
_NB = _WIN * _N1
_Q = 4 * _T
_O = _WIN * _WIN
_BIG = 3.0e38
_HI = jax.lax.Precision.HIGHEST


def _ci_of(t):
    return min(max(t - _WIN // 2, 0), _N1 - _WIN)


def _chw_to_hwc_kernel(ye_ref, xe_ref, yo_ref, xo_ref):
    yo_ref[...] = jnp.transpose(ye_ref[...], (1, 0))
    xo_ref[...] = jnp.transpose(xe_ref[...], (1, 0))


def _knn_band_kernel(ye_ref, xe_ref, idx_ref, sc_ref, vd_ref):
    u = pl.program_id(0)
    cj = jnp.clip(u - _WIN // 2, 0, _N1 - _WIN)
    c2u = 2 * u

    def yslice(i, j):
        yb = ye_ref[i:i + _M1, pl.ds(c2u + j, 2), :]
        return yb.reshape(_Q, _CE)

    def xslice(i, j):
        xb = xe_ref[i:i + _N1, pl.ds(cj + j, _WIN), :]
        return xb.reshape(_NB, _CE)

    db = jnp.zeros((_Q, _NB), jnp.float32)
    ynorm = jnp.zeros((_Q, 1), jnp.float32)
    xn8 = jnp.zeros((8, _NB), jnp.float32)
    ones8 = jnp.full((8, _CE), 1.0, jnp.float32)
    for i in range(_PS):
        for j in range(_PS):
            Yij = yslice(i, j)
            Xij = xslice(i, j)
            db = db + jax.lax.dot_general(
                Yij, Xij, (((1,), (1,)), ((), ())),
                preferred_element_type=jnp.float32)
            ynorm = ynorm + jnp.sum(Yij * Yij, axis=1, keepdims=True)
            xn8 = xn8 + jax.lax.dot_general(
                ones8, Xij * Xij, (((1,), (1,)), ((), ())),
                preferred_element_type=jnp.float32, precision=_HI)
    db = ynorm + xn8[0:1] - 2.0 * db

    rows = []
    for t in range(_T):
        ci = _ci_of(t)
        rows.append(db[4 * t:4 * t + 4, ci * _WIN:ci * _WIN + _O])
    dw = jnp.concatenate(rows, axis=0) + 1e-5

    lane = jax.lax.broadcasted_iota(jnp.int32, (_Q, _O), 1)
    r4 = jax.lax.broadcasted_iota(jnp.int32, (_Q, 1), 0)
    ci_vec = jnp.clip(r4 // 4 - _WIN // 2, 0, _N1 - _WIN)

    cur = dw
    lis = []
    for k in range(_K):
        mv = jnp.min(cur, axis=1, keepdims=True)
        o = jnp.min(jnp.where(cur == mv, lane, _O), axis=1, keepdims=True)
        sc_ref[:, 0, :, k:k + 1] = jnp.exp(
            mv * jnp.float32(-0.1)).reshape(_M1, 2, 1)
        idx_ref[0, :, k:k + 1] = (ci_vec + o // _WIN) * _N1 + cj + o % _WIN
        lis.append((ci_vec + o // _WIN) * _WIN + o % _WIN)
        cur = jnp.where(lane == o, _BIG, cur)

    li5 = jnp.concatenate(lis, axis=0)
    lane360_5 = jax.lax.broadcasted_iota(jnp.int32, (_K * _Q, _NB), 1)
    oh5 = (li5 == lane360_5).astype(jnp.float32)
    vd = jnp.zeros((_K * _Q, _CE), jnp.float32)
    for i in range(_PS):
        for j in range(_PS):
            g = jax.lax.dot_general(
                oh5, xslice(i, j), (((1,), (0,)), ((), ())),
                preferred_element_type=jnp.float32)
            Yij = yslice(i, j)
            vd = vd + jnp.abs(
                jnp.concatenate([Yij] * _K, axis=0) - g)
    vd_ref[...] = vd.reshape(_K, _M1, 1, 2, _CE)


def _box_kernel(v_ref, b_ref):
    v = v_ref[...]

    def shift(di, dj):
        cols = []
        if dj > 0:
            cols.append(jnp.zeros((97, _M1, dj), jnp.float32))
        cols.append(v)
        if dj < 2:
            cols.append(jnp.zeros((97, _M1, 2 - dj), jnp.float32))
        sw = jnp.concatenate(cols, axis=2) if len(cols) > 1 else cols[0]
        rows = []
        if di > 0:
            rows.append(jnp.zeros((97, di, 64), jnp.float32))
        rows.append(sw)
        if di < 2:
            rows.append(jnp.zeros((97, 2 - di, 64), jnp.float32))
        return jnp.concatenate(rows, axis=1) if len(rows) > 1 else rows[0]

    acc = shift(0, 0)
    for di in range(3):
        for dj in range(3):
            if di or dj:
                acc = acc + shift(di, dj)
    b_ref[...] = acc


def kernel(xe, ye):
    yo, xo = pl.pallas_call(
        _chw_to_hwc_kernel,
        out_shape=[
            jax.ShapeDtypeStruct((64 * 64, _CE), jnp.float32),
            jax.ShapeDtypeStruct((32 * 32, _CE), jnp.float32),
        ],
    )(ye[0].reshape(_CE, 64 * 64), xe[0].reshape(_CE, 32 * 32))
    ye_hwc = yo.reshape(64, 64, _CE)
    xe_hwc = xo.reshape(32, 32, _CE)

    idx4, sc4, vd5 = pl.pallas_call(
        _knn_band_kernel,
        grid=(_T,),
        in_specs=[
            pl.BlockSpec((64, 64, _CE), lambda u: (0, 0, 0)),
            pl.BlockSpec((32, 32, _CE), lambda u: (0, 0, 0)),
        ],
        out_specs=[
            pl.BlockSpec((1, _Q, _K), lambda u: (u, 0, 0)),
            pl.BlockSpec((_M1, 1, 2, _K), lambda u: (0, u, 0, 0)),
            pl.BlockSpec((_K, _M1, 1, 2, _CE), lambda u: (0, 0, u, 0, 0)),
        ],
        out_shape=[
            jax.ShapeDtypeStruct((_T, _Q, _K), jnp.int32),
            jax.ShapeDtypeStruct((_M1, _T, 2, _K), jnp.float32),
            jax.ShapeDtypeStruct((_K, _M1, _T, 2, _CE), jnp.float32),
        ],
        compiler_params=pltpu.CompilerParams(
            dimension_semantics=("arbitrary",)),
    )(ye_hwc, xe_hwc)

    idx_k = idx4.reshape(_T, _M1, 2, _K).transpose(1, 0, 2, 3) \
                .reshape(1, _M1 * _M1, _K)

    vs = sc4.reshape(_M1, _M1, _K).transpose(2, 0, 1)
    vd = vd5.reshape(_K, _M1 * _M1, _CE).transpose(0, 2, 1) \
            .reshape(_K * _CE, _M1, _M1)
    v_all = jnp.concatenate([vs, vd], axis=0)

    box = pl.pallas_call(
        _box_kernel,
        grid=(5,),
        in_specs=[pl.BlockSpec((97, _M1, _M1), lambda i: (i, 0, 0))],
        out_specs=pl.BlockSpec((97, 64, 64), lambda i: (i, 0, 0)),
        out_shape=jax.ShapeDtypeStruct((485, 64, 64), jnp.float32),
        compiler_params=pltpu.CompilerParams(
            dimension_semantics=("arbitrary",)),
    )(v_all)

    up = jnp.broadcast_to(box[:, :, None, :, None], (485, 64, 2, 64, 2)) \
            .reshape(485, 128, 128)
    sc_im = up[None, :_K]
    diff_im = up[None, _K:]
    return (sc_im, idx_k, diff_im)


# speedup vs baseline: 1.0518x; 1.0518x over previous
import jax
import jax.numpy as jnp
from jax.experimental import pallas as pl
from jax.experimental.pallas import tpu as pltpu

_K = 5
_PS = 3
_WIN = 12
_N1 = 30
_M1 = 62
_T = 31
_CE = 96
_NB = _WIN * _N1
_Q = 4 * _T
_O = _WIN * _WIN
_BIG = 3.0e38
_HI = jax.lax.Precision.HIGHEST


def _ci_of(t):
    return min(max(t - _WIN // 2, 0), _N1 - _WIN)


def _knn_band_kernel(ye_ref, xe_ref, idx_ref, sc_ref, vd_ref):
    u = pl.program_id(0)
    cj = jnp.clip(u - _WIN // 2, 0, _N1 - _WIN)
    c2u = 2 * u

    def yslice(i, j):
        yb = ye_ref[i:i + _M1, pl.ds(c2u + j, 2), :]
        return yb.reshape(_Q, _CE)

    def xslice(i, j):
        xb = xe_ref[i:i + _N1, pl.ds(cj + j, _WIN), :]
        return xb.reshape(_NB, _CE)

    db = jnp.zeros((_Q, _NB), jnp.float32)
    ynorm = jnp.zeros((_Q, 1), jnp.float32)
    xn8 = jnp.zeros((8, _NB), jnp.float32)
    ones8 = jnp.full((8, _CE), 1.0, jnp.float32)
    for i in range(_PS):
        for j in range(_PS):
            Yij = yslice(i, j)
            Xij = xslice(i, j)
            db = db + jax.lax.dot_general(
                Yij, Xij, (((1,), (1,)), ((), ())),
                preferred_element_type=jnp.float32)
            ynorm = ynorm + jnp.sum(Yij * Yij, axis=1, keepdims=True)
            xn8 = xn8 + jax.lax.dot_general(
                ones8, Xij * Xij, (((1,), (1,)), ((), ())),
                preferred_element_type=jnp.float32, precision=_HI)
    db = ynorm + xn8[0:1] - 2.0 * db

    rows = []
    for t in range(_T):
        ci = _ci_of(t)
        rows.append(db[4 * t:4 * t + 4, ci * _WIN:ci * _WIN + _O])
    dw = jnp.concatenate(rows, axis=0) + 1e-5

    lane = jax.lax.broadcasted_iota(jnp.int32, (_Q, _O), 1)
    r4 = jax.lax.broadcasted_iota(jnp.int32, (_Q, 1), 0)
    ci_vec = jnp.clip(r4 // 4 - _WIN // 2, 0, _N1 - _WIN)

    cur = dw
    lis = []
    for k in range(_K):
        mv = jnp.min(cur, axis=1, keepdims=True)
        o = jnp.min(jnp.where(cur == mv, lane, _O), axis=1, keepdims=True)
        sc_ref[0, :, k:k + 1] = jnp.exp(mv * jnp.float32(-0.1))
        idx_ref[0, :, k:k + 1] = (ci_vec + o // _WIN) * _N1 + cj + o % _WIN
        lis.append((ci_vec + o // _WIN) * _WIN + o % _WIN)
        cur = jnp.where(lane == o, _BIG, cur)

    li5 = jnp.concatenate(lis, axis=0)
    lane360_5 = jax.lax.broadcasted_iota(jnp.int32, (_K * _Q, _NB), 1)
    oh5 = (li5 == lane360_5).astype(jnp.float32)
    vd = jnp.zeros((_K * _Q, _CE), jnp.float32)
    for i in range(_PS):
        for j in range(_PS):
            g = jax.lax.dot_general(
                oh5, xslice(i, j), (((1,), (0,)), ((), ())),
                preferred_element_type=jnp.float32)
            Yij = yslice(i, j)
            vd = vd + jnp.abs(
                jnp.concatenate([Yij] * _K, axis=0) - g)
    for k in range(_K):
        vd_ref[0, k] = vd[k * _Q:(k + 1) * _Q]


def _box_kernel(v_ref, b_ref):
    v = v_ref[...]

    def shift(di, dj):
        cols = []
        if dj > 0:
            cols.append(jnp.zeros((97, _M1, dj), jnp.float32))
        cols.append(v)
        if dj < 2:
            cols.append(jnp.zeros((97, _M1, 2 - dj), jnp.float32))
        sw = jnp.concatenate(cols, axis=2) if len(cols) > 1 else cols[0]
        rows = []
        if di > 0:
            rows.append(jnp.zeros((97, di, 64), jnp.float32))
        rows.append(sw)
        if di < 2:
            rows.append(jnp.zeros((97, 2 - di, 64), jnp.float32))
        return jnp.concatenate(rows, axis=1) if len(rows) > 1 else rows[0]

    acc = shift(0, 0)
    for di in range(3):
        for dj in range(3):
            if di or dj:
                acc = acc + shift(di, dj)
    b_ref[...] = acc


def kernel(xe, ye):
    xe_hwc = jnp.transpose(xe[0], (1, 2, 0))
    ye_hwc = jnp.transpose(ye[0], (1, 2, 0))

    idx4, sc4, vd5 = pl.pallas_call(
        _knn_band_kernel,
        grid=(_T,),
        in_specs=[
            pl.BlockSpec((64, 64, _CE), lambda u: (0, 0, 0)),
            pl.BlockSpec((32, 32, _CE), lambda u: (0, 0, 0)),
        ],
        out_specs=[
            pl.BlockSpec((1, _Q, _K), lambda u: (u, 0, 0)),
            pl.BlockSpec((1, _Q, _K), lambda u: (u, 0, 0)),
            pl.BlockSpec((1, _K, _Q, _CE), lambda u: (u, 0, 0, 0)),
        ],
        out_shape=[
            jax.ShapeDtypeStruct((_T, _Q, _K), jnp.int32),
            jax.ShapeDtypeStruct((_T, _Q, _K), jnp.float32),
            jax.ShapeDtypeStruct((_T, _K, _Q, _CE), jnp.float32),
        ],
        compiler_params=pltpu.CompilerParams(
            dimension_semantics=("arbitrary",)),
    )(ye_hwc, xe_hwc)

    idx_k = idx4.reshape(_T, _M1, 2, _K).transpose(1, 0, 2, 3) \
                .reshape(1, _M1 * _M1, _K)

    vs = sc4.reshape(_T, _M1, 2, _K).transpose(3, 1, 0, 2).reshape(_K, _M1, _M1)
    vd = vd5.reshape(_T, _K, _M1, 2, _CE).transpose(1, 4, 2, 0, 3) \
            .reshape(_K * _CE, _M1, _M1)
    v_all = jnp.concatenate([vs, vd], axis=0)

    box = pl.pallas_call(
        _box_kernel,
        grid=(5,),
        in_specs=[pl.BlockSpec((97, _M1, _M1), lambda i: (i, 0, 0))],
        out_specs=pl.BlockSpec((97, 64, 64), lambda i: (i, 0, 0)),
        out_shape=jax.ShapeDtypeStruct((485, 64, 64), jnp.float32),
        compiler_params=pltpu.CompilerParams(
            dimension_semantics=("arbitrary",)),
    )(v_all)

    up = jnp.broadcast_to(box[:, :, None, :, None], (485, 64, 2, 64, 2)) \
            .reshape(485, 128, 128)
    sc_im = up[None, :_K]
    diff_im = up[None, _K:]
    return (sc_im, idx_k, diff_im)
